# baseline (speedup 1.0000x reference)
import jax
import jax.numpy as jnp
import numpy as np
from jax.experimental import pallas as pl

N = 10000
HEADS = 8
HID = 8


def _gat_conv(x, edge_index, W, a_src, a_dst, b, heads, out_ch):
    n = x.shape[0]
    loop = jnp.arange(n, dtype=edge_index.dtype)
    src = jnp.concatenate([edge_index[0], loop])
    dst = jnp.concatenate([edge_index[1], loop])
    h = (x @ W).reshape(n, heads, out_ch)
    alpha_src = jnp.sum(h * a_src[None, :, :], axis=-1)
    alpha_dst = jnp.sum(h * a_dst[None, :, :], axis=-1)
    e = alpha_src[src] + alpha_dst[dst]
    e = jax.nn.leaky_relu(e, negative_slope=0.2)
    m = jax.ops.segment_max(e, dst, num_segments=n)
    m = jnp.where(jnp.isfinite(m), m, 0.0)
    ex = jnp.exp(e - m[dst])
    s = jax.ops.segment_sum(ex, dst, num_segments=n)
    alpha = ex / (s[dst] + 1e-16)
    msg = h[src] * alpha[:, :, None]
    out = jax.ops.segment_sum(msg, dst, num_segments=n)
    return out.reshape(n, heads * out_ch) + b


def kernel(x, edge_index, W1, a_src1, a_dst1, b1, W2, a_src2, a_dst2, b2):
    h = _gat_conv(x, edge_index, W1, a_src1, a_dst1, b1, HEADS, HID)
    h = jax.nn.elu(h)
    out = _gat_conv(h, edge_index, W2, a_src2, a_dst2, b2, 1, 2)
    return out


# speedup vs baseline: 46.7557x; 46.7557x over previous
import functools

import jax
import jax.numpy as jnp
from jax import lax
from jax.experimental import pallas as pl
from jax.experimental.pallas import tpu as pltpu
from jax.experimental.pallas import tpu_sc as plsc

N = 10000
DIM = 128
HID = 8
HEADS = 8
NCLS = 2
F1 = HEADS * HID

NC = 2
NS = 16
LANES = 16

ROW1 = 80
ROW2 = 16
CH = 80
PADN = 10240
BIG = 100.0


def _l1_prologue_body(x_ref, w1_ref, msrc_ref, mdst_ref, st_ref, dt_ref, c_ref):
    h = jnp.dot(x_ref[...], w1_ref[...], preferred_element_type=jnp.float32)
    as1 = jnp.dot(h, msrc_ref[...], preferred_element_type=jnp.float32)
    ad1 = jnp.dot(h, mdst_ref[...], preferred_element_type=jnp.float32)
    z8 = jnp.zeros((N, 8), jnp.float32)
    st_ref[...] = jnp.concatenate([h, as1, z8], axis=1)
    dt_ref[...] = jnp.concatenate([ad1, z8], axis=1)
    c8 = (jnp.max(as1, axis=0, keepdims=True)
          + jnp.max(ad1, axis=0, keepdims=True))
    c_ref[...] = jnp.concatenate([c8, jnp.full((1, 8), BIG, jnp.float32)],
                                 axis=1)


def _head_expand(m):
    row = lax.broadcasted_iota(jnp.int32, (HEADS, F1), 0)
    col = lax.broadcasted_iota(jnp.int32, (HEADS, F1), 1)
    r = (col // HID == row).astype(jnp.float32)
    return jnp.dot(m, r, preferred_element_type=jnp.float32)


def _l1_finalize_body(acc_ref, st_ref, dt_ref, c_ref, b1_ref, w2_ref,
                      asv_ref, adv_ref, st2_ref, dt2_ref, c2_ref):
    a = acc_ref[0, 0:N, :] + acc_ref[1, 0:N, :]
    u = a[:, 0:F1]
    se = a[:, F1:F1 + HEADS]
    h1 = st_ref[:, 0:F1]
    as1 = st_ref[:, F1:F1 + HEADS]
    ad1 = dt_ref[:, 0:HEADS]
    c = c_ref[:, 0:HEADS]
    t = as1 + ad1
    e = jnp.maximum(t, 0.2 * t)
    exl = jnp.exp(e - c)
    s = se + exl + 1e-16
    out1 = ((u + _head_expand(exl) * h1) * _head_expand(1.0 / s)
            + b1_ref[...])
    h1e = jnp.where(out1 > 0, out1, jnp.exp(jnp.minimum(out1, 0.0)) - 1.0)
    h2 = jnp.dot(h1e, w2_ref[...], preferred_element_type=jnp.float32)
    as2 = jnp.dot(h2, asv_ref[...], preferred_element_type=jnp.float32)
    ad2 = jnp.dot(h2, adv_ref[...], preferred_element_type=jnp.float32)
    ones = jnp.ones((N, 1), jnp.float32)
    st2_ref[...] = jnp.concatenate(
        [h2, ones, as2, jnp.zeros((N, 12), jnp.float32)], axis=1)
    dt2_ref[...] = jnp.concatenate(
        [jnp.zeros((N, 3), jnp.float32), ad2, jnp.zeros((N, 12), jnp.float32)],
        axis=1)
    c2 = (jnp.max(as2, axis=0, keepdims=True)
          + jnp.max(ad2, axis=0, keepdims=True))
    big = jnp.full((1, 3), BIG, jnp.float32)
    big12 = jnp.full((1, 12), BIG, jnp.float32)
    c2_ref[...] = jnp.concatenate([big, c2, big12], axis=1)


def _l2_finalize_body(acc_ref, st2_ref, dt2_ref, c2_ref, b2_ref, out_ref):
    a = acc_ref[0, 0:N, :] + acc_ref[1, 0:N, :]
    u = a[:, 0:NCLS]
    se = a[:, NCLS:NCLS + 1]
    h2 = st2_ref[:, 0:NCLS]
    as2 = st2_ref[:, 3:4]
    ad2 = dt2_ref[:, 3:4]
    c2 = c2_ref[:, 3:4]
    t = as2 + ad2
    e = jnp.maximum(t, 0.2 * t)
    exl = jnp.exp(e - c2)
    s = se + exl + 1e-16
    out_ref[...] = (u + exl * h2) / s + b2_ref[...]


def _sc_edges_l1(src_hbm, dst_hbm, st_hbm, dt_hbm, c_hbm, z_hbm, out_hbm,
                 acc, idxs_v, idxd_v, srows_v, drows_v, stage_v, exbuf_v,
                 cvec_v, sem0, sem1):
    cid = lax.axis_index("c")
    sid = lax.axis_index("s")
    wid = cid * NS + sid
    rows = PADN // NS
    r0 = sid * rows
    pltpu.sync_copy(z_hbm.at[pl.ds(r0, rows)], acc.at[pl.ds(r0, rows)])
    pltpu.sync_copy(c_hbm, cvec_v)
    plsc.subcore_barrier()

    cv = cvec_v[0, :]
    iot = lax.iota(jnp.int32, LANES)
    colpat = iot >> 3
    epw = src_hbm.shape[0] // (NC * NS)
    nch = epw // CH
    base = wid * epw

    def chunk(tch, carry):
        off = base + tch * CH
        pltpu.sync_copy(src_hbm.at[pl.ds(off, CH)], idxs_v)
        pltpu.sync_copy(dst_hbm.at[pl.ds(off, CH)], idxd_v)
        cp0 = pltpu.async_copy(st_hbm.at[idxs_v], srows_v, sem0)
        cp1 = pltpu.async_copy(dt_hbm.at[idxd_v], drows_v, sem1)
        cp0.wait()
        cp1.wait()

        def edge(i, c2):
            va = srows_v[i, pl.ds(F1, LANES)]
            vd = drows_v[i, pl.ds(0, LANES)]
            t = va + vd
            e = jnp.maximum(t, 0.2 * t)
            ex = jnp.exp(e - cv)
            stage_v[i, pl.ds(F1, LANES)] = ex
            exbuf_v[pl.ds(i * LANES, LANES)] = ex
            ridx = i * LANES + colpat
            for j in range(4):
                bj = plsc.load_gather(exbuf_v, [ridx + 2 * j])
                hv = srows_v[i, pl.ds(LANES * j, LANES)]
                stage_v[i, pl.ds(LANES * j, LANES)] = hv * bj
            return c2

        lax.fori_loop(0, CH, edge, 0)
        pltpu.sync_copy(stage_v, acc.at[idxd_v], add=True)
        return carry

    lax.fori_loop(0, nch, chunk, 0)
    plsc.subcore_barrier()
    pltpu.sync_copy(acc.at[pl.ds(r0, rows)],
                    out_hbm.at[cid, pl.ds(r0, rows)])


def _sc_edges_l2(src_hbm, dst_hbm, st_hbm, dt_hbm, c_hbm, z_hbm, out_hbm,
                 acc, idxs_v, idxd_v, srows_v, drows_v, stage_v, exbuf_v,
                 cvec_v, sem0, sem1):
    cid = lax.axis_index("c")
    sid = lax.axis_index("s")
    wid = cid * NS + sid
    rows = PADN // NS
    r0 = sid * rows
    pltpu.sync_copy(z_hbm.at[pl.ds(r0, rows)], acc.at[pl.ds(r0, rows)])
    pltpu.sync_copy(c_hbm, cvec_v)
    plsc.subcore_barrier()

    cv = cvec_v[0, :]
    col3 = jnp.full((LANES,), 3, jnp.int32)
    epw = src_hbm.shape[0] // (NC * NS)
    nch = epw // CH
    base = wid * epw

    def chunk(tch, carry):
        off = base + tch * CH
        pltpu.sync_copy(src_hbm.at[pl.ds(off, CH)], idxs_v)
        pltpu.sync_copy(dst_hbm.at[pl.ds(off, CH)], idxd_v)
        cp0 = pltpu.async_copy(st_hbm.at[idxs_v], srows_v, sem0)
        cp1 = pltpu.async_copy(dt_hbm.at[idxd_v], drows_v, sem1)
        cp0.wait()
        cp1.wait()

        def edge(i, c2):
            vs = srows_v[i, pl.ds(0, LANES)]
            vd = drows_v[i, pl.ds(0, LANES)]
            w = vs + vd
            e = jnp.maximum(w, 0.2 * w)
            exv = jnp.exp(e - cv)
            exbuf_v[pl.ds(i * LANES, LANES)] = exv
            exb = plsc.load_gather(exbuf_v, [i * LANES + col3])
            stage_v[i, pl.ds(0, LANES)] = vs * exb
            return c2

        lax.fori_loop(0, CH, edge, 0)
        pltpu.sync_copy(stage_v, acc.at[idxd_v], add=True)
        return carry

    lax.fori_loop(0, nch, chunk, 0)
    plsc.subcore_barrier()
    pltpu.sync_copy(acc.at[pl.ds(r0, rows)],
                    out_hbm.at[cid, pl.ds(r0, rows)])


def kernel(x, edge_index, W1, a_src1, a_dst1, b1, W2, a_src2, a_dst2, b2):
    f32 = jnp.float32
    src = edge_index[0]
    dst = edge_index[1]

    eye = jnp.eye(HEADS, dtype=f32)
    msrc = (a_src1[:, :, None] * eye[:, None, :]).reshape(F1, HEADS)
    mdst = (a_dst1[:, :, None] * eye[:, None, :]).reshape(F1, HEADS)

    st1, dt1, c1 = pl.pallas_call(
        _l1_prologue_body,
        out_shape=[
            jax.ShapeDtypeStruct((N, ROW1), f32),
            jax.ShapeDtypeStruct((N, 16), f32),
            jax.ShapeDtypeStruct((1, 16), f32),
        ],
    )(x, W1, msrc, mdst)

    mesh = plsc.VectorSubcoreMesh(core_axis_name="c", subcore_axis_name="s")
    zeros1 = jnp.zeros((PADN, ROW1), f32)
    acc1 = pl.kernel(
        _sc_edges_l1,
        out_type=jax.ShapeDtypeStruct((NC, PADN, ROW1), f32),
        mesh=mesh,
        compiler_params=pltpu.CompilerParams(needs_layout_passes=False, use_tc_tiling_on_sc=False),
        scratch_types=[
            pltpu.VMEM_SHARED((PADN, ROW1), f32),
            pltpu.VMEM((CH,), jnp.int32),
            pltpu.VMEM((CH,), jnp.int32),
            pltpu.VMEM((CH, ROW1), f32),
            pltpu.VMEM((CH, 16), f32),
            pltpu.VMEM((CH, ROW1), f32),
            pltpu.VMEM((CH * LANES,), f32),
            pltpu.VMEM((1, 16), f32),
            pltpu.SemaphoreType.DMA,
            pltpu.SemaphoreType.DMA,
        ],
    )(src, dst, st1, dt1, c1, zeros1)

    st2, dt2, c2 = pl.pallas_call(
        _l1_finalize_body,
        out_shape=[
            jax.ShapeDtypeStruct((N, ROW2), f32),
            jax.ShapeDtypeStruct((N, ROW2), f32),
            jax.ShapeDtypeStruct((1, 16), f32),
        ],
    )(acc1, st1, dt1, c1, b1, W2, a_src2.reshape(NCLS, 1),
      a_dst2.reshape(NCLS, 1))

    zeros2 = jnp.zeros((PADN, ROW2), f32)
    acc2 = pl.kernel(
        _sc_edges_l2,
        out_type=jax.ShapeDtypeStruct((NC, PADN, ROW2), f32),
        mesh=mesh,
        compiler_params=pltpu.CompilerParams(needs_layout_passes=False, use_tc_tiling_on_sc=False),
        scratch_types=[
            pltpu.VMEM_SHARED((PADN, ROW2), f32),
            pltpu.VMEM((CH,), jnp.int32),
            pltpu.VMEM((CH,), jnp.int32),
            pltpu.VMEM((CH, ROW2), f32),
            pltpu.VMEM((CH, ROW2), f32),
            pltpu.VMEM((CH, ROW2), f32),
            pltpu.VMEM((CH * LANES,), f32),
            pltpu.VMEM((1, 16), f32),
            pltpu.SemaphoreType.DMA,
            pltpu.SemaphoreType.DMA,
        ],
    )(src, dst, st2, dt2, c2, zeros2)

    out = pl.pallas_call(
        _l2_finalize_body,
        out_shape=jax.ShapeDtypeStruct((N, NCLS), f32),
    )(acc2, st2, dt2, c2, b2)
    return out


# speedup vs baseline: 190.0647x; 4.0651x over previous
import jax
import jax.numpy as jnp
from jax import lax
from jax.experimental import pallas as pl
from jax.experimental.pallas import tpu as pltpu
from jax.experimental.pallas import tpu_sc as plsc

N = 10000
DIM = 128
HID = 8
HEADS = 8
NCLS = 2
F1 = HEADS * HID

NC = 2
NS = 16
NW = NC * NS
LANES = 16

ROW1 = 80
ROW2 = 16
SROW2 = 32
CHK = 100
NCHK = 100
PADN = 10240
BIG = 100.0


def _perm_mat():
    o = lax.broadcasted_iota(jnp.int32, (F1, F1), 0)
    j = lax.broadcasted_iota(jnp.int32, (F1, F1), 1)
    return (o == 8 * (j % 8) + j // 8).astype(jnp.float32)


def _l1_prologue_body(x_ref, w1_ref, msrc_ref, mdst_ref, st_ref, dt_ref, c_ref):
    h = jnp.dot(x_ref[...], w1_ref[...], preferred_element_type=jnp.float32)
    as1 = jnp.dot(h, msrc_ref[...], preferred_element_type=jnp.float32)
    ad1 = jnp.dot(h, mdst_ref[...], preferred_element_type=jnp.float32)
    hp = jnp.dot(h, _perm_mat(), preferred_element_type=jnp.float32)
    st_ref[...] = jnp.concatenate([hp, as1, as1], axis=1)
    dt_ref[...] = jnp.concatenate([ad1, ad1], axis=1)
    c8 = (jnp.max(as1, axis=0, keepdims=True)
          + jnp.max(ad1, axis=0, keepdims=True))
    c_ref[...] = jnp.concatenate([c8, c8], axis=1)


def _head_expand(m):
    row = lax.broadcasted_iota(jnp.int32, (HEADS, F1), 0)
    col = lax.broadcasted_iota(jnp.int32, (HEADS, F1), 1)
    r = (col % HID == row).astype(jnp.float32)
    return jnp.dot(m, r, preferred_element_type=jnp.float32)


def _l1_finalize_body(acc_ref, st_ref, dt_ref, c_ref, b1_ref, w2_ref,
                      asv_ref, adv_ref, st2_ref, dt2_ref, c2_ref):
    a = acc_ref[0, 0:N, :] + acc_ref[1, 0:N, :]
    u = a[:, 0:F1]
    se = a[:, F1:F1 + HEADS]
    h1 = st_ref[:, 0:F1]
    as1 = st_ref[:, F1:F1 + HEADS]
    ad1 = dt_ref[:, 0:HEADS]
    c = c_ref[:, 0:HEADS]
    t = as1 + ad1
    e = jnp.maximum(t, 0.2 * t)
    exl = jnp.exp(e - c)
    s = se + exl + 1e-16
    out1 = ((u + _head_expand(exl) * h1) * _head_expand(1.0 / s)
            + b1_ref[...])
    h1e = jnp.where(out1 > 0, out1, jnp.exp(jnp.minimum(out1, 0.0)) - 1.0)
    h2 = jnp.dot(h1e, w2_ref[...], preferred_element_type=jnp.float32)
    as2 = jnp.dot(h2, asv_ref[...], preferred_element_type=jnp.float32)
    ad2 = jnp.dot(h2, adv_ref[...], preferred_element_type=jnp.float32)
    ones = jnp.ones((N, 1), jnp.float32)
    as2b = jnp.broadcast_to(as2, (N, LANES))
    ad2b = jnp.broadcast_to(ad2, (N, LANES))
    st2_ref[...] = jnp.concatenate(
        [h2, ones, jnp.zeros((N, 13), jnp.float32), as2b], axis=1)
    dt2_ref[...] = ad2b
    c2 = (jnp.max(as2, axis=0, keepdims=True)
          + jnp.max(ad2, axis=0, keepdims=True))
    c2_ref[...] = jnp.broadcast_to(c2, (1, LANES))


def _l2_finalize_body(acc_ref, st2_ref, dt2_ref, c2_ref, b2_ref, out_ref):
    a = acc_ref[0, 0:N, :] + acc_ref[1, 0:N, :]
    u = a[:, 0:NCLS]
    se = a[:, NCLS:NCLS + 1]
    h2 = st2_ref[:, 0:NCLS]
    as2 = st2_ref[:, LANES:LANES + 1]
    ad2 = dt2_ref[:, 3:4]
    c2 = c2_ref[:, 3:4]
    t = as2 + ad2
    e = jnp.maximum(t, 0.2 * t)
    exl = jnp.exp(e - c2)
    s = se + exl + 1e-16
    out_ref[...] = (u + exl * h2) / s + b2_ref[...]


def _sc_edges_l1(src_hbm, dst_hbm, st_hbm, dt_hbm, c_hbm, z_hbm, out_hbm,
                 acc, sidx_v, didx_v, srows_v, drows_v, stage_v,
                 cvec_v, semg0, semg1, semc0, semc1):
    cid = lax.axis_index("c")
    sid = lax.axis_index("s")
    wid = cid * NS + sid
    rows = PADN // NS
    r0 = sid * rows
    pltpu.sync_copy(z_hbm.at[pl.ds(r0, rows)], acc.at[pl.ds(r0, rows)])
    pltpu.sync_copy(c_hbm, cvec_v)
    pltpu.sync_copy(src_hbm.at[wid], sidx_v)
    pltpu.sync_copy(dst_hbm.at[wid], didx_v)
    plsc.subcore_barrier()

    cv = cvec_v[0, :]
    semg = (semg0, semg1)
    semc = (semc0, semc1)

    def gathers(t, p):
        return (
            pltpu.make_async_copy(st_hbm.at[sidx_v.at[t]], srows_v.at[p],
                                  semg[p]),
            pltpu.make_async_copy(dt_hbm.at[didx_v.at[t]], drows_v.at[p],
                                  semg[p]),
        )

    def scatter(t, p):
        return pltpu.make_async_copy(stage_v.at[p], acc.at[didx_v.at[t]],
                                     semc[p])

    for g in gathers(0, 0):
        g.start()
    for g in gathers(1, 1):
        g.start()

    def pair(t2, carry):
        for p in (0, 1):
            t = 2 * t2 + p
            for g in gathers(t, p):
                g.wait()

            @pl.when(t2 > 0)
            def _():
                scatter(t - 2, p).wait()

            def edge(i):
                va = srows_v[p, i, pl.ds(F1, LANES)]
                vd = drows_v[p, i, pl.ds(0, LANES)]
                tt = va + vd
                e = jnp.maximum(tt, 0.2 * tt)
                exd = jnp.exp(e - cv)
                stage_v[p, i, pl.ds(F1, LANES)] = exd
                for j in range(4):
                    hv = srows_v[p, i, pl.ds(LANES * j, LANES)]
                    stage_v[p, i, pl.ds(LANES * j, LANES)] = hv * exd

            plsc.parallel_loop(0, CHK, unroll=8)(edge)

            @pl.when(t + 2 < NCHK)
            def _():
                for g in gathers(t + 2, p):
                    g.start()

            scatter(t, p).start(add=True)
        return carry

    lax.fori_loop(0, NCHK // 2, pair, 0)
    scatter(NCHK - 2, 0).wait()
    scatter(NCHK - 1, 1).wait()
    plsc.subcore_barrier()
    pltpu.sync_copy(acc.at[pl.ds(r0, rows)],
                    out_hbm.at[cid, pl.ds(r0, rows)])


def _sc_edges_l2(src_hbm, dst_hbm, st_hbm, dt_hbm, c_hbm, z_hbm, out_hbm,
                 acc, sidx_v, didx_v, srows_v, drows_v, stage_v,
                 cvec_v, semg0, semg1, semc0, semc1):
    cid = lax.axis_index("c")
    sid = lax.axis_index("s")
    wid = cid * NS + sid
    rows = PADN // NS
    r0 = sid * rows
    pltpu.sync_copy(z_hbm.at[pl.ds(r0, rows)], acc.at[pl.ds(r0, rows)])
    pltpu.sync_copy(c_hbm, cvec_v)
    pltpu.sync_copy(src_hbm.at[wid], sidx_v)
    pltpu.sync_copy(dst_hbm.at[wid], didx_v)
    plsc.subcore_barrier()

    cv = cvec_v[0, :]
    semg = (semg0, semg1)
    semc = (semc0, semc1)

    def gathers(t, p):
        return (
            pltpu.make_async_copy(st_hbm.at[sidx_v.at[t]], srows_v.at[p],
                                  semg[p]),
            pltpu.make_async_copy(dt_hbm.at[didx_v.at[t]], drows_v.at[p],
                                  semg[p]),
        )

    def scatter(t, p):
        return pltpu.make_async_copy(stage_v.at[p], acc.at[didx_v.at[t]],
                                     semc[p])

    for g in gathers(0, 0):
        g.start()
    for g in gathers(1, 1):
        g.start()

    def pair(t2, carry):
        for p in (0, 1):
            t = 2 * t2 + p
            for g in gathers(t, p):
                g.wait()

            @pl.when(t2 > 0)
            def _():
                scatter(t - 2, p).wait()

            def edge(i):
                va = srows_v[p, i, pl.ds(LANES, LANES)]
                vd = drows_v[p, i, pl.ds(0, LANES)]
                w = va + vd
                e = jnp.maximum(w, 0.2 * w)
                exd = jnp.exp(e - cv)
                vm = srows_v[p, i, pl.ds(0, LANES)]
                stage_v[p, i, pl.ds(0, LANES)] = vm * exd

            plsc.parallel_loop(0, CHK, unroll=8)(edge)

            @pl.when(t + 2 < NCHK)
            def _():
                for g in gathers(t + 2, p):
                    g.start()

            scatter(t, p).start(add=True)
        return carry

    lax.fori_loop(0, NCHK // 2, pair, 0)
    scatter(NCHK - 2, 0).wait()
    scatter(NCHK - 1, 1).wait()
    plsc.subcore_barrier()
    pltpu.sync_copy(acc.at[pl.ds(r0, rows)],
                    out_hbm.at[cid, pl.ds(r0, rows)])


def kernel(x, edge_index, W1, a_src1, a_dst1, b1, W2, a_src2, a_dst2, b2):
    f32 = jnp.float32
    src = edge_index[0].reshape(NW, NCHK, CHK)
    dst = edge_index[1].reshape(NW, NCHK, CHK)

    eye = jnp.eye(HEADS, dtype=f32)
    msrc = (a_src1[:, :, None] * eye[:, None, :]).reshape(F1, HEADS)
    mdst = (a_dst1[:, :, None] * eye[:, None, :]).reshape(F1, HEADS)
    b1p = b1.reshape(HEADS, HID).T.reshape(F1)
    W2p = W2.reshape(HEADS, HID, NCLS).transpose(1, 0, 2).reshape(F1, NCLS)

    st1, dt1, c1 = pl.pallas_call(
        _l1_prologue_body,
        out_shape=[
            jax.ShapeDtypeStruct((N, ROW1), f32),
            jax.ShapeDtypeStruct((N, 16), f32),
            jax.ShapeDtypeStruct((1, 16), f32),
        ],
    )(x, W1, msrc, mdst)

    mesh = plsc.VectorSubcoreMesh(core_axis_name="c", subcore_axis_name="s")
    sc_params = pltpu.CompilerParams(needs_layout_passes=False,
                                     use_tc_tiling_on_sc=False)
    zeros1 = jnp.zeros((PADN, ROW1), f32)
    acc1 = pl.kernel(
        _sc_edges_l1,
        out_type=jax.ShapeDtypeStruct((NC, PADN, ROW1), f32),
        mesh=mesh,
        compiler_params=sc_params,
        scratch_types=[
            pltpu.VMEM_SHARED((PADN, ROW1), f32),
            pltpu.VMEM((NCHK, CHK), jnp.int32),
            pltpu.VMEM((NCHK, CHK), jnp.int32),
            pltpu.VMEM((2, CHK, ROW1), f32),
            pltpu.VMEM((2, CHK, 16), f32),
            pltpu.VMEM((2, CHK, ROW1), f32),
            pltpu.VMEM((1, 16), f32),
            pltpu.SemaphoreType.DMA,
            pltpu.SemaphoreType.DMA,
            pltpu.SemaphoreType.DMA,
            pltpu.SemaphoreType.DMA,
        ],
    )(src, dst, st1, dt1, c1, zeros1)

    st2, dt2, c2 = pl.pallas_call(
        _l1_finalize_body,
        out_shape=[
            jax.ShapeDtypeStruct((N, SROW2), f32),
            jax.ShapeDtypeStruct((N, ROW2), f32),
            jax.ShapeDtypeStruct((1, 16), f32),
        ],
    )(acc1, st1, dt1, c1, b1p, W2p, a_src2.reshape(NCLS, 1),
      a_dst2.reshape(NCLS, 1))

    zeros2 = jnp.zeros((PADN, ROW2), f32)
    acc2 = pl.kernel(
        _sc_edges_l2,
        out_type=jax.ShapeDtypeStruct((NC, PADN, ROW2), f32),
        mesh=mesh,
        compiler_params=sc_params,
        scratch_types=[
            pltpu.VMEM_SHARED((PADN, ROW2), f32),
            pltpu.VMEM((NCHK, CHK), jnp.int32),
            pltpu.VMEM((NCHK, CHK), jnp.int32),
            pltpu.VMEM((2, CHK, SROW2), f32),
            pltpu.VMEM((2, CHK, ROW2), f32),
            pltpu.VMEM((2, CHK, ROW2), f32),
            pltpu.VMEM((1, 16), f32),
            pltpu.SemaphoreType.DMA,
            pltpu.SemaphoreType.DMA,
            pltpu.SemaphoreType.DMA,
            pltpu.SemaphoreType.DMA,
        ],
    )(src, dst, st2, dt2, c2, zeros2)

    out = pl.pallas_call(
        _l2_finalize_body,
        out_shape=jax.ShapeDtypeStruct((N, NCLS), f32),
    )(acc2, st2, dt2, c2, b2)
    return out


# speedup vs baseline: 195.5084x; 1.0286x over previous
import jax
import jax.numpy as jnp
from jax import lax
from jax.experimental import pallas as pl
from jax.experimental.pallas import tpu as pltpu
from jax.experimental.pallas import tpu_sc as plsc

N = 10000
DIM = 128
HID = 8
HEADS = 8
NCLS = 2
F1 = HEADS * HID

NC = 2
NS = 16
NW = NC * NS
LANES = 16

ROW1 = 80
ROW2 = 16
SROW2 = 32
CHK = 125
NCHK = 80
PADN = 10240
BIG = 100.0


def _l1_prologue_body(x_ref, w1_ref, msrc_ref, mdst_ref, st_ref, dt_ref, c_ref):
    h = jnp.dot(x_ref[...], w1_ref[...], preferred_element_type=jnp.float32)
    as1 = jnp.dot(h, msrc_ref[...], preferred_element_type=jnp.float32)
    ad1 = jnp.dot(h, mdst_ref[...], preferred_element_type=jnp.float32)
    st_ref[...] = jnp.concatenate([h, as1, as1], axis=1)
    dt_ref[...] = jnp.concatenate([ad1, ad1], axis=1)
    c8 = (jnp.max(as1, axis=0, keepdims=True)
          + jnp.max(ad1, axis=0, keepdims=True))
    c_ref[...] = jnp.concatenate([c8, c8], axis=1)


def _head_expand(m):
    row = lax.broadcasted_iota(jnp.int32, (HEADS, F1), 0)
    col = lax.broadcasted_iota(jnp.int32, (HEADS, F1), 1)
    r = (col % HID == row).astype(jnp.float32)
    return jnp.dot(m, r, preferred_element_type=jnp.float32)


def _l1_finalize_body(acc_ref, st_ref, dt_ref, c_ref, b1_ref, w2_ref,
                      asv_ref, adv_ref, st2_ref, dt2_ref, c2_ref):
    a = acc_ref[0, 0:N, :] + acc_ref[1, 0:N, :]
    u = a[:, 0:F1]
    se = a[:, F1:F1 + HEADS]
    h1 = st_ref[:, 0:F1]
    as1 = st_ref[:, F1:F1 + HEADS]
    ad1 = dt_ref[:, 0:HEADS]
    c = c_ref[:, 0:HEADS]
    t = as1 + ad1
    e = jnp.maximum(t, 0.2 * t)
    exl = jnp.exp(e - c)
    s = se + exl + 1e-16
    out1 = ((u + _head_expand(exl) * h1) * _head_expand(1.0 / s)
            + b1_ref[...])
    h1e = jnp.where(out1 > 0, out1, jnp.exp(jnp.minimum(out1, 0.0)) - 1.0)
    h2 = jnp.dot(h1e, w2_ref[...], preferred_element_type=jnp.float32)
    as2 = jnp.dot(h2, asv_ref[...], preferred_element_type=jnp.float32)
    ad2 = jnp.dot(h2, adv_ref[...], preferred_element_type=jnp.float32)
    ones = jnp.ones((N, 1), jnp.float32)
    as2b = jnp.broadcast_to(as2, (N, LANES))
    ad2b = jnp.broadcast_to(ad2, (N, LANES))
    st2_ref[...] = jnp.concatenate(
        [h2, ones, jnp.zeros((N, 13), jnp.float32), as2b], axis=1)
    dt2_ref[...] = ad2b
    c2 = (jnp.max(as2, axis=0, keepdims=True)
          + jnp.max(ad2, axis=0, keepdims=True))
    c2_ref[...] = jnp.broadcast_to(c2, (1, LANES))


def _l2_finalize_body(acc_ref, st2_ref, dt2_ref, c2_ref, b2_ref, out_ref):
    a = acc_ref[0, 0:N, :] + acc_ref[1, 0:N, :]
    u = a[:, 0:NCLS]
    se = a[:, NCLS:NCLS + 1]
    h2 = st2_ref[:, 0:NCLS]
    as2 = st2_ref[:, LANES:LANES + 1]
    ad2 = dt2_ref[:, 3:4]
    c2 = c2_ref[:, 3:4]
    t = as2 + ad2
    e = jnp.maximum(t, 0.2 * t)
    exl = jnp.exp(e - c2)
    s = se + exl + 1e-16
    out_ref[...] = (u + exl * h2) / s + b2_ref[...]


def _sc_edges_l1(src_hbm, dst_hbm, st_hbm, dt_hbm, c_hbm, z_hbm, out_hbm,
                 acc, sidx_v, didx_v, srows_v, drows_v, stage_v,
                 cvec_v, semg0, semg1, semc0, semc1):
    cid = lax.axis_index("c")
    sid = lax.axis_index("s")
    wid = cid * NS + sid
    rows = PADN // NS
    r0 = sid * rows
    pltpu.sync_copy(z_hbm.at[pl.ds(r0, rows)], acc.at[pl.ds(r0, rows)])
    pltpu.sync_copy(c_hbm, cvec_v)
    pltpu.sync_copy(src_hbm.at[wid], sidx_v)
    pltpu.sync_copy(dst_hbm.at[wid], didx_v)
    plsc.subcore_barrier()

    cv = cvec_v[0, :]
    semg = (semg0, semg1)
    semc = (semc0, semc1)

    def gathers(t, p):
        return (
            pltpu.make_async_copy(st_hbm.at[sidx_v.at[t]], srows_v.at[p],
                                  semg[p]),
            pltpu.make_async_copy(dt_hbm.at[didx_v.at[t]], drows_v.at[p],
                                  semg[p]),
        )

    def scatter(t, p):
        return pltpu.make_async_copy(stage_v.at[p], acc.at[didx_v.at[t]],
                                     semc[p])

    for g in gathers(0, 0):
        g.start()
    for g in gathers(1, 1):
        g.start()

    def pair(t2, carry):
        for p in (0, 1):
            t = 2 * t2 + p
            for g in gathers(t, p):
                g.wait()

            @pl.when(t2 > 0)
            def _():
                scatter(t - 2, p).wait()

            def edge(i):
                va = srows_v[p, i, pl.ds(F1, LANES)]
                vd = drows_v[p, i, pl.ds(0, LANES)]
                tt = va + vd
                e = jnp.maximum(tt, 0.2 * tt)
                exd = jnp.exp(e - cv)
                stage_v[p, i, pl.ds(F1, LANES)] = exd
                for j in range(4):
                    hv = srows_v[p, i, pl.ds(LANES * j, LANES)]
                    stage_v[p, i, pl.ds(LANES * j, LANES)] = hv * exd

            plsc.parallel_loop(0, CHK, unroll=8)(edge)

            @pl.when(t + 2 < NCHK)
            def _():
                for g in gathers(t + 2, p):
                    g.start()

            scatter(t, p).start(add=True)
        return carry

    lax.fori_loop(0, NCHK // 2, pair, 0)
    scatter(NCHK - 2, 0).wait()
    scatter(NCHK - 1, 1).wait()
    plsc.subcore_barrier()
    pltpu.sync_copy(acc.at[pl.ds(r0, rows)],
                    out_hbm.at[cid, pl.ds(r0, rows)])


def _sc_edges_l2(src_hbm, dst_hbm, st_hbm, dt_hbm, c_hbm, z_hbm, out_hbm,
                 acc, sidx_v, didx_v, srows_v, drows_v, stage_v,
                 cvec_v, semg0, semg1, semc0, semc1):
    cid = lax.axis_index("c")
    sid = lax.axis_index("s")
    wid = cid * NS + sid
    rows = PADN // NS
    r0 = sid * rows
    pltpu.sync_copy(z_hbm.at[pl.ds(r0, rows)], acc.at[pl.ds(r0, rows)])
    pltpu.sync_copy(c_hbm, cvec_v)
    pltpu.sync_copy(src_hbm.at[wid], sidx_v)
    pltpu.sync_copy(dst_hbm.at[wid], didx_v)
    plsc.subcore_barrier()

    cv = cvec_v[0, :]
    semg = (semg0, semg1)
    semc = (semc0, semc1)

    def gathers(t, p):
        return (
            pltpu.make_async_copy(st_hbm.at[sidx_v.at[t]], srows_v.at[p],
                                  semg[p]),
            pltpu.make_async_copy(dt_hbm.at[didx_v.at[t]], drows_v.at[p],
                                  semg[p]),
        )

    def scatter(t, p):
        return pltpu.make_async_copy(stage_v.at[p], acc.at[didx_v.at[t]],
                                     semc[p])

    for g in gathers(0, 0):
        g.start()
    for g in gathers(1, 1):
        g.start()

    def pair(t2, carry):
        for p in (0, 1):
            t = 2 * t2 + p
            for g in gathers(t, p):
                g.wait()

            @pl.when(t2 > 0)
            def _():
                scatter(t - 2, p).wait()

            def edge(i):
                va = srows_v[p, i, pl.ds(LANES, LANES)]
                vd = drows_v[p, i, pl.ds(0, LANES)]
                w = va + vd
                e = jnp.maximum(w, 0.2 * w)
                exd = jnp.exp(e - cv)
                vm = srows_v[p, i, pl.ds(0, LANES)]
                stage_v[p, i, pl.ds(0, LANES)] = vm * exd

            plsc.parallel_loop(0, CHK, unroll=8)(edge)

            @pl.when(t + 2 < NCHK)
            def _():
                for g in gathers(t + 2, p):
                    g.start()

            scatter(t, p).start(add=True)
        return carry

    lax.fori_loop(0, NCHK // 2, pair, 0)
    scatter(NCHK - 2, 0).wait()
    scatter(NCHK - 1, 1).wait()
    plsc.subcore_barrier()
    pltpu.sync_copy(acc.at[pl.ds(r0, rows)],
                    out_hbm.at[cid, pl.ds(r0, rows)])


def kernel(x, edge_index, W1, a_src1, a_dst1, b1, W2, a_src2, a_dst2, b2):
    f32 = jnp.float32
    src = edge_index[0].reshape(NW, NCHK, CHK)
    dst = edge_index[1].reshape(NW, NCHK, CHK)

    eye = jnp.eye(HEADS, dtype=f32)
    msrc = (a_src1[:, :, None] * eye[:, None, :]).reshape(F1, HEADS)
    mdst = (a_dst1[:, :, None] * eye[:, None, :]).reshape(F1, HEADS)
    pidx = 8 * (jnp.arange(F1) % HEADS) + jnp.arange(F1) // HEADS
    W1p = W1[:, pidx]
    msrcp = msrc[pidx, :]
    mdstp = mdst[pidx, :]
    b1p = b1.reshape(HEADS, HID).T.reshape(F1)
    W2p = W2.reshape(HEADS, HID, NCLS).transpose(1, 0, 2).reshape(F1, NCLS)

    st1, dt1, c1 = pl.pallas_call(
        _l1_prologue_body,
        out_shape=[
            jax.ShapeDtypeStruct((N, ROW1), f32),
            jax.ShapeDtypeStruct((N, 16), f32),
            jax.ShapeDtypeStruct((1, 16), f32),
        ],
    )(x, W1p, msrcp, mdstp)

    mesh = plsc.VectorSubcoreMesh(core_axis_name="c", subcore_axis_name="s")
    sc_params = pltpu.CompilerParams(needs_layout_passes=False,
                                     use_tc_tiling_on_sc=False)
    zeros1 = jnp.zeros((PADN, ROW1), f32)
    acc1 = pl.kernel(
        _sc_edges_l1,
        out_type=jax.ShapeDtypeStruct((NC, PADN, ROW1), f32),
        mesh=mesh,
        compiler_params=sc_params,
        scratch_types=[
            pltpu.VMEM_SHARED((PADN, ROW1), f32),
            pltpu.VMEM((NCHK, CHK), jnp.int32),
            pltpu.VMEM((NCHK, CHK), jnp.int32),
            pltpu.VMEM((2, CHK, ROW1), f32),
            pltpu.VMEM((2, CHK, 16), f32),
            pltpu.VMEM((2, CHK, ROW1), f32),
            pltpu.VMEM((1, 16), f32),
            pltpu.SemaphoreType.DMA,
            pltpu.SemaphoreType.DMA,
            pltpu.SemaphoreType.DMA,
            pltpu.SemaphoreType.DMA,
        ],
    )(src, dst, st1, dt1, c1, zeros1)

    st2, dt2, c2 = pl.pallas_call(
        _l1_finalize_body,
        out_shape=[
            jax.ShapeDtypeStruct((N, SROW2), f32),
            jax.ShapeDtypeStruct((N, ROW2), f32),
            jax.ShapeDtypeStruct((1, 16), f32),
        ],
    )(acc1, st1, dt1, c1, b1p, W2p, a_src2.reshape(NCLS, 1),
      a_dst2.reshape(NCLS, 1))

    zeros2 = jnp.zeros((PADN, ROW2), f32)
    acc2 = pl.kernel(
        _sc_edges_l2,
        out_type=jax.ShapeDtypeStruct((NC, PADN, ROW2), f32),
        mesh=mesh,
        compiler_params=sc_params,
        scratch_types=[
            pltpu.VMEM_SHARED((PADN, ROW2), f32),
            pltpu.VMEM((NCHK, CHK), jnp.int32),
            pltpu.VMEM((NCHK, CHK), jnp.int32),
            pltpu.VMEM((2, CHK, SROW2), f32),
            pltpu.VMEM((2, CHK, ROW2), f32),
            pltpu.VMEM((2, CHK, ROW2), f32),
            pltpu.VMEM((1, 16), f32),
            pltpu.SemaphoreType.DMA,
            pltpu.SemaphoreType.DMA,
            pltpu.SemaphoreType.DMA,
            pltpu.SemaphoreType.DMA,
        ],
    )(src, dst, st2, dt2, c2, zeros2)

    out = pl.pallas_call(
        _l2_finalize_body,
        out_shape=jax.ShapeDtypeStruct((N, NCLS), f32),
    )(acc2, st2, dt2, c2, b2)
    return out


# speedup vs baseline: 196.0784x; 1.0029x over previous
import jax
import jax.numpy as jnp
from jax import lax
from jax.experimental import pallas as pl
from jax.experimental.pallas import tpu as pltpu
from jax.experimental.pallas import tpu_sc as plsc

N = 10000
DIM = 128
HID = 8
HEADS = 8
NCLS = 2
F1 = HEADS * HID

NC = 2
NS = 16
NW = NC * NS
LANES = 16

ROW1 = 80
ROW2 = 16
SROW2 = 32
CHK = 125
NCHK = 80
PADN = 10240
BIG = 100.0


def _l1_prologue_body(x_ref, w1_ref, msrc_ref, mdst_ref, st_ref, dt_ref, c_ref):
    h = jnp.dot(x_ref[...], w1_ref[...], preferred_element_type=jnp.float32)
    as1 = jnp.dot(h, msrc_ref[...], preferred_element_type=jnp.float32)
    ad1 = jnp.dot(h, mdst_ref[...], preferred_element_type=jnp.float32)
    st_ref[...] = jnp.concatenate([h, as1, as1], axis=1)
    dt_ref[...] = jnp.concatenate([ad1, ad1], axis=1)
    c8 = (jnp.max(as1, axis=0, keepdims=True)
          + jnp.max(ad1, axis=0, keepdims=True))
    c_ref[...] = jnp.concatenate([c8, c8], axis=1)


def _head_expand(m):
    row = lax.broadcasted_iota(jnp.int32, (HEADS, F1), 0)
    col = lax.broadcasted_iota(jnp.int32, (HEADS, F1), 1)
    r = (col % HID == row).astype(jnp.float32)
    return jnp.dot(m, r, preferred_element_type=jnp.float32)


def _l1_finalize_body(acc_ref, st_ref, dt_ref, c_ref, b1_ref, w2_ref,
                      asv_ref, adv_ref, st2_ref, dt2_ref, c2_ref):
    a = acc_ref[0, 0:N, :] + acc_ref[1, 0:N, :]
    u = a[:, 0:F1]
    se = a[:, F1:F1 + HEADS]
    h1 = st_ref[:, 0:F1]
    as1 = st_ref[:, F1:F1 + HEADS]
    ad1 = dt_ref[:, 0:HEADS]
    c = c_ref[:, 0:HEADS]
    t = as1 + ad1
    e = jnp.maximum(t, 0.2 * t)
    exl = jnp.exp(e - c)
    s = se + exl + 1e-16
    out1 = ((u + _head_expand(exl) * h1) * _head_expand(1.0 / s)
            + b1_ref[...])
    h1e = jnp.where(out1 > 0, out1, jnp.exp(jnp.minimum(out1, 0.0)) - 1.0)
    h2 = jnp.dot(h1e, w2_ref[...], preferred_element_type=jnp.float32)
    as2 = jnp.dot(h2, asv_ref[...], preferred_element_type=jnp.float32)
    ad2 = jnp.dot(h2, adv_ref[...], preferred_element_type=jnp.float32)
    ones = jnp.ones((N, 1), jnp.float32)
    as2b = jnp.broadcast_to(as2, (N, LANES))
    ad2b = jnp.broadcast_to(ad2, (N, LANES))
    st2_ref[...] = jnp.concatenate(
        [h2, ones, jnp.zeros((N, 13), jnp.float32), as2b], axis=1)
    dt2_ref[...] = ad2b
    c2 = (jnp.max(as2, axis=0, keepdims=True)
          + jnp.max(ad2, axis=0, keepdims=True))
    c2_ref[...] = jnp.broadcast_to(c2, (1, LANES))


def _l2_finalize_body(acc_ref, st2_ref, dt2_ref, c2_ref, b2_ref, out_ref):
    a = acc_ref[0, 0:N, :] + acc_ref[1, 0:N, :]
    u = a[:, 0:NCLS]
    se = a[:, NCLS:NCLS + 1]
    h2 = st2_ref[:, 0:NCLS]
    as2 = st2_ref[:, LANES:LANES + 1]
    ad2 = dt2_ref[:, 3:4]
    c2 = c2_ref[:, 3:4]
    t = as2 + ad2
    e = jnp.maximum(t, 0.2 * t)
    exl = jnp.exp(e - c2)
    s = se + exl + 1e-16
    out_ref[...] = (u + exl * h2) / s + b2_ref[...]


def _sc_edges_l1(src_hbm, dst_hbm, st_hbm, dt_hbm, c_hbm, out_hbm,
                 acc, sidx_v, didx_v, srows_v, drows_v, stage_v,
                 cvec_v, semg0, semg1, semc0, semc1):
    cid = lax.axis_index("c")
    sid = lax.axis_index("s")
    wid = cid * NS + sid
    rows = PADN // NS
    r0 = sid * rows

    def zrow(i):
        for j in range(ROW1 // LANES):
            stage_v[0, i, pl.ds(LANES * j, LANES)] = jnp.zeros(
                (LANES,), jnp.float32)

    plsc.parallel_loop(0, CHK, unroll=8)(zrow)
    for k in range(rows // 80):
        pltpu.sync_copy(stage_v.at[0, pl.ds(0, 80)],
                        acc.at[pl.ds(r0 + 80 * k, 80)])
    pltpu.sync_copy(c_hbm, cvec_v)
    pltpu.sync_copy(src_hbm.at[wid], sidx_v)
    pltpu.sync_copy(dst_hbm.at[wid], didx_v)
    plsc.subcore_barrier()

    cv = cvec_v[0, :]
    semg = (semg0, semg1)
    semc = (semc0, semc1)

    def gathers(t, p):
        return (
            pltpu.make_async_copy(st_hbm.at[sidx_v.at[t]], srows_v.at[p],
                                  semg[p]),
            pltpu.make_async_copy(dt_hbm.at[didx_v.at[t]], drows_v.at[p],
                                  semg[p]),
        )

    def scatter(t, p):
        return pltpu.make_async_copy(stage_v.at[p], acc.at[didx_v.at[t]],
                                     semc[p])

    for g in gathers(0, 0):
        g.start()
    for g in gathers(1, 1):
        g.start()

    def pair(t2, carry):
        for p in (0, 1):
            t = 2 * t2 + p
            for g in gathers(t, p):
                g.wait()

            @pl.when(t2 > 0)
            def _():
                scatter(t - 2, p).wait()

            def edge(i):
                va = srows_v[p, i, pl.ds(F1, LANES)]
                vd = drows_v[p, i, pl.ds(0, LANES)]
                tt = va + vd
                e = jnp.maximum(tt, 0.2 * tt)
                exd = jnp.exp(e - cv)
                stage_v[p, i, pl.ds(F1, LANES)] = exd
                for j in range(4):
                    hv = srows_v[p, i, pl.ds(LANES * j, LANES)]
                    stage_v[p, i, pl.ds(LANES * j, LANES)] = hv * exd

            plsc.parallel_loop(0, CHK, unroll=8)(edge)

            @pl.when(t + 2 < NCHK)
            def _():
                for g in gathers(t + 2, p):
                    g.start()

            scatter(t, p).start(add=True)
        return carry

    lax.fori_loop(0, NCHK // 2, pair, 0)
    scatter(NCHK - 2, 0).wait()
    scatter(NCHK - 1, 1).wait()
    plsc.subcore_barrier()
    pltpu.sync_copy(acc.at[pl.ds(r0, rows)],
                    out_hbm.at[cid, pl.ds(r0, rows)])


def _sc_edges_l2(src_hbm, dst_hbm, st_hbm, dt_hbm, c_hbm, out_hbm,
                 acc, sidx_v, didx_v, srows_v, drows_v, stage_v,
                 cvec_v, semg0, semg1, semc0, semc1):
    cid = lax.axis_index("c")
    sid = lax.axis_index("s")
    wid = cid * NS + sid
    rows = PADN // NS
    r0 = sid * rows

    def zrow(i):
        stage_v[0, i, pl.ds(0, LANES)] = jnp.zeros((LANES,), jnp.float32)

    plsc.parallel_loop(0, CHK, unroll=8)(zrow)
    for k in range(rows // 80):
        pltpu.sync_copy(stage_v.at[0, pl.ds(0, 80)],
                        acc.at[pl.ds(r0 + 80 * k, 80)])
    pltpu.sync_copy(c_hbm, cvec_v)
    pltpu.sync_copy(src_hbm.at[wid], sidx_v)
    pltpu.sync_copy(dst_hbm.at[wid], didx_v)
    plsc.subcore_barrier()

    cv = cvec_v[0, :]
    semg = (semg0, semg1)
    semc = (semc0, semc1)

    def gathers(t, p):
        return (
            pltpu.make_async_copy(st_hbm.at[sidx_v.at[t]], srows_v.at[p],
                                  semg[p]),
            pltpu.make_async_copy(dt_hbm.at[didx_v.at[t]], drows_v.at[p],
                                  semg[p]),
        )

    def scatter(t, p):
        return pltpu.make_async_copy(stage_v.at[p], acc.at[didx_v.at[t]],
                                     semc[p])

    for g in gathers(0, 0):
        g.start()
    for g in gathers(1, 1):
        g.start()

    def pair(t2, carry):
        for p in (0, 1):
            t = 2 * t2 + p
            for g in gathers(t, p):
                g.wait()

            @pl.when(t2 > 0)
            def _():
                scatter(t - 2, p).wait()

            def edge(i):
                va = srows_v[p, i, pl.ds(LANES, LANES)]
                vd = drows_v[p, i, pl.ds(0, LANES)]
                w = va + vd
                e = jnp.maximum(w, 0.2 * w)
                exd = jnp.exp(e - cv)
                vm = srows_v[p, i, pl.ds(0, LANES)]
                stage_v[p, i, pl.ds(0, LANES)] = vm * exd

            plsc.parallel_loop(0, CHK, unroll=16)(edge)

            @pl.when(t + 2 < NCHK)
            def _():
                for g in gathers(t + 2, p):
                    g.start()

            scatter(t, p).start(add=True)
        return carry

    lax.fori_loop(0, NCHK // 2, pair, 0)
    scatter(NCHK - 2, 0).wait()
    scatter(NCHK - 1, 1).wait()
    plsc.subcore_barrier()
    pltpu.sync_copy(acc.at[pl.ds(r0, rows)],
                    out_hbm.at[cid, pl.ds(r0, rows)])


def kernel(x, edge_index, W1, a_src1, a_dst1, b1, W2, a_src2, a_dst2, b2):
    f32 = jnp.float32
    src = edge_index[0].reshape(NW, NCHK, CHK)
    dst = edge_index[1].reshape(NW, NCHK, CHK)

    eye = jnp.eye(HEADS, dtype=f32)
    msrc = (a_src1[:, :, None] * eye[:, None, :]).reshape(F1, HEADS)
    mdst = (a_dst1[:, :, None] * eye[:, None, :]).reshape(F1, HEADS)
    pidx = 8 * (jnp.arange(F1) % HEADS) + jnp.arange(F1) // HEADS
    W1p = W1[:, pidx]
    msrcp = msrc[pidx, :]
    mdstp = mdst[pidx, :]
    b1p = b1.reshape(HEADS, HID).T.reshape(F1)
    W2p = W2.reshape(HEADS, HID, NCLS).transpose(1, 0, 2).reshape(F1, NCLS)

    st1, dt1, c1 = pl.pallas_call(
        _l1_prologue_body,
        out_shape=[
            jax.ShapeDtypeStruct((N, ROW1), f32),
            jax.ShapeDtypeStruct((N, 16), f32),
            jax.ShapeDtypeStruct((1, 16), f32),
        ],
    )(x, W1p, msrcp, mdstp)

    mesh = plsc.VectorSubcoreMesh(core_axis_name="c", subcore_axis_name="s")
    sc_params = pltpu.CompilerParams(needs_layout_passes=False,
                                     use_tc_tiling_on_sc=False)
    acc1 = pl.kernel(
        _sc_edges_l1,
        out_type=jax.ShapeDtypeStruct((NC, PADN, ROW1), f32),
        mesh=mesh,
        compiler_params=sc_params,
        scratch_types=[
            pltpu.VMEM_SHARED((PADN, ROW1), f32),
            pltpu.VMEM((NCHK, CHK), jnp.int32),
            pltpu.VMEM((NCHK, CHK), jnp.int32),
            pltpu.VMEM((2, CHK, ROW1), f32),
            pltpu.VMEM((2, CHK, 16), f32),
            pltpu.VMEM((2, CHK, ROW1), f32),
            pltpu.VMEM((1, 16), f32),
            pltpu.SemaphoreType.DMA,
            pltpu.SemaphoreType.DMA,
            pltpu.SemaphoreType.DMA,
            pltpu.SemaphoreType.DMA,
        ],
    )(src, dst, st1, dt1, c1)

    st2, dt2, c2 = pl.pallas_call(
        _l1_finalize_body,
        out_shape=[
            jax.ShapeDtypeStruct((N, SROW2), f32),
            jax.ShapeDtypeStruct((N, ROW2), f32),
            jax.ShapeDtypeStruct((1, 16), f32),
        ],
    )(acc1, st1, dt1, c1, b1p, W2p, a_src2.reshape(NCLS, 1),
      a_dst2.reshape(NCLS, 1))

    acc2 = pl.kernel(
        _sc_edges_l2,
        out_type=jax.ShapeDtypeStruct((NC, PADN, ROW2), f32),
        mesh=mesh,
        compiler_params=sc_params,
        scratch_types=[
            pltpu.VMEM_SHARED((PADN, ROW2), f32),
            pltpu.VMEM((NCHK, CHK), jnp.int32),
            pltpu.VMEM((NCHK, CHK), jnp.int32),
            pltpu.VMEM((2, CHK, SROW2), f32),
            pltpu.VMEM((2, CHK, ROW2), f32),
            pltpu.VMEM((2, CHK, ROW2), f32),
            pltpu.VMEM((1, 16), f32),
            pltpu.SemaphoreType.DMA,
            pltpu.SemaphoreType.DMA,
            pltpu.SemaphoreType.DMA,
            pltpu.SemaphoreType.DMA,
        ],
    )(src, dst, st2, dt2, c2)

    out = pl.pallas_call(
        _l2_finalize_body,
        out_shape=jax.ShapeDtypeStruct((N, NCLS), f32),
    )(acc2, st2, dt2, c2, b2)
    return out
